# baseline (speedup 1.0000x reference)
import functools

import jax
import jax.numpy as jnp
from jax import lax
from jax.experimental import pallas as pl
from jax.experimental.pallas import tpu as pltpu
from jax.experimental.pallas import tpu_sc as plsc

NN = 10000
NE = 320000
DF = 128
DP = 144
NC = 2
NS = 16
NW = NC * NS
EPW = NE // NW
CH = 80
NCHUNK = EPW // CH
NPAD = 10240
RPT = NPAD // NS


def _make_agg(width):
  mesh = plsc.VectorSubcoreMesh(
      core_axis_name="c", subcore_axis_name="s", num_cores=NC, num_subcores=NS)
  lanes_per_row = width // 16

  def body(feat, srcs, dsts, out, sidx, didx0, didx1,
           rows0, rows1, acc, gsem0, gsem1, ssem0, ssem1, dsem0, dsem1):
    c = lax.axis_index("c")
    s = lax.axis_index("s")
    wid = c * NS + s
    ebase = wid * EPW

    pltpu.sync_copy(srcs.at[pl.ds(ebase, EPW)], sidx)

    def dload(i, didx, dsem):
      return pltpu.async_copy(dsts.at[pl.ds(ebase + i * CH, CH)], didx, dsem)

    def dwait(didx, dsem):
      pltpu.make_async_copy(dsts.at[pl.ds(0, CH)], didx, dsem).wait()

    zv = jnp.zeros((16,), jnp.float32)

    def zr(i, carry):
      rows0[i // lanes_per_row, pl.ds((i % lanes_per_row) * 16, 16)] = zv
      return carry

    lax.fori_loop(0, CH * lanes_per_row, zr, 0)
    rbase = s * RPT

    def zc(i, carry):
      pltpu.async_copy(rows0, acc.at[pl.ds(rbase + i * CH, CH)], ssem0)
      return carry

    lax.fori_loop(0, RPT // CH, zc, 0)

    def zw(i, carry):
      pltpu.make_async_copy(rows0, acc.at[pl.ds(rbase, CH)], ssem0).wait()
      return carry

    lax.fori_loop(0, RPT // CH, zw, 0)
    plsc.subcore_barrier()

    def gather(i, rows, gsem):
      return pltpu.async_copy(feat.at[sidx.at[pl.ds(i * CH, CH)]], rows, gsem)

    def gwait(rows, gsem):
      pltpu.make_async_copy(feat.at[sidx.at[pl.ds(0, CH)]], rows, gsem).wait()

    def swait(rows, didx, ssem):
      pltpu.make_async_copy(rows, acc.at[didx], ssem).wait()

    gather(0, rows0, gsem0)
    dload(0, didx0, dsem0)

    def pair(p, carry):
      i0 = 2 * p
      i1 = i0 + 1

      @pl.when(p > 0)
      def _():
        swait(rows1, didx1, ssem1)

      gather(i1, rows1, gsem1)
      dload(i1, didx1, dsem1)
      gwait(rows0, gsem0)
      dwait(didx0, dsem0)
      pltpu.async_copy(rows0, acc.at[didx0], ssem0, add=True)
      swait(rows0, didx0, ssem0)
      gather(i1 + 1, rows0, gsem0)
      dload(i1 + 1, didx0, dsem0)
      gwait(rows1, gsem1)
      dwait(didx1, dsem1)
      pltpu.async_copy(rows1, acc.at[didx1], ssem1, add=True)
      return carry

    lax.fori_loop(0, NCHUNK // 2, pair, 0)
    swait(rows1, didx1, ssem1)
    gwait(rows0, gsem0)
    dwait(didx0, dsem0)
    pltpu.sync_copy(rows0, acc.at[didx0], add=True)
    plsc.subcore_barrier()

    pltpu.sync_copy(acc.at[pl.ds(rbase, RPT)], out.at[c, pl.ds(rbase, RPT)])

  return pl.kernel(
      body,
      out_type=jax.ShapeDtypeStruct((NC, NPAD, width), jnp.float32),
      mesh=mesh,
      scratch_types=[
          pltpu.VMEM((EPW,), jnp.int32),
          pltpu.VMEM((CH,), jnp.int32),
          pltpu.VMEM((CH,), jnp.int32),
          pltpu.VMEM((CH, width), jnp.float32),
          pltpu.VMEM((CH, width), jnp.float32),
          pltpu.VMEM_SHARED((NPAD, width), jnp.float32),
          pltpu.SemaphoreType.DMA,
          pltpu.SemaphoreType.DMA,
          pltpu.SemaphoreType.DMA,
          pltpu.SemaphoreType.DMA,
          pltpu.SemaphoreType.DMA,
          pltpu.SemaphoreType.DMA,
      ],
      compiler_params=pltpu.CompilerParams(use_tc_tiling_on_sc=(width % 128 == 0)),
      name=f"sage_sc_agg_{width}",
  )


_DN = (((1,), (1,)), ((), ()))


def _pre_l_body(x_ref, wl_ref, xlp_ref):
  xlp_ref[:, :DF] = lax.dot_general(
      x_ref[...], wl_ref[...], _DN, preferred_element_type=jnp.float32)
  col = lax.broadcasted_iota(jnp.int32, (NN, DP - DF), 1)
  xlp_ref[:, DF:] = jnp.where(col == 0, 1.0, 0.0)


def _pre_r_body(x_ref, wr_ref, b_ref, xr_ref):
  xr_ref[...] = lax.dot_general(
      x_ref[...], wr_ref[...], _DN, preferred_element_type=jnp.float32
  ) + b_ref[...]


def _mid_a_body(acc_ref, xr_ref, h1_ref, dinv_ref):
  accs = acc_ref[0, :NN, :] + acc_ref[1, :NN, :]
  dinv = 1.0 / jnp.maximum(accs[:, DF:DF + 1], 1.0)
  h1_ref[...] = jnp.maximum(accs[:, :DF] * dinv + xr_ref[...], 0.0)
  dinv_ref[...] = dinv


def _mid_l_body(h1_ref, wl_ref, h1l_ref):
  h1l_ref[...] = lax.dot_general(
      h1_ref[...], wl_ref[...], _DN, preferred_element_type=jnp.float32)


def _mid_r_body(h1_ref, wr_ref, b_ref, h1r_ref):
  h1r_ref[...] = lax.dot_general(
      h1_ref[...], wr_ref[...], _DN, preferred_element_type=jnp.float32
  ) + b_ref[...]


def _post_body(acc_ref, dinv_ref, h1r_ref, out_ref):
  accs = acc_ref[0, :NN, :] + acc_ref[1, :NN, :]
  out_ref[...] = accs * dinv_ref[...] + h1r_ref[...]


def kernel(x, edge_index, W1l, b1, W1r, W2l, b2, W2r):
  src = edge_index[0]
  dst = edge_index[1]
  f32 = jnp.float32
  sds = jax.ShapeDtypeStruct

  xlp = pl.pallas_call(
      _pre_l_body, out_shape=sds((NN, DP), f32))(x, W1l)
  xr = pl.pallas_call(
      _pre_r_body, out_shape=sds((NN, DF), f32))(x, W1r, b1.reshape(1, DF))

  acc1 = _make_agg(DP)(xlp, src, dst)

  h1, dinv = pl.pallas_call(
      _mid_a_body,
      out_shape=[sds((NN, DF), f32), sds((NN, 1), f32)],
  )(acc1, xr)
  h1l = pl.pallas_call(
      _mid_l_body, out_shape=sds((NN, DF), f32))(h1, W2l)
  h1r = pl.pallas_call(
      _mid_r_body, out_shape=sds((NN, DF), f32))(h1, W2r, b2.reshape(1, DF))

  acc2 = _make_agg(DF)(h1l, src, dst)

  h2 = pl.pallas_call(
      _post_body, out_shape=sds((NN, DF), f32))(acc2, dinv, h1r)

  return h2


# speedup vs baseline: 1.0046x; 1.0046x over previous
import functools

import jax
import jax.numpy as jnp
from jax import lax
from jax.experimental import pallas as pl
from jax.experimental.pallas import tpu as pltpu
from jax.experimental.pallas import tpu_sc as plsc

NN = 10000
NE = 320000
DF = 128
DP = 144
NC = 2
NS = 16
NW = NC * NS
EPW = NE // NW
CH = 80
NCHUNK = EPW // CH
NPAD = 10240
RPT = NPAD // NS


def _make_agg(width):
  mesh = plsc.VectorSubcoreMesh(
      core_axis_name="c", subcore_axis_name="s", num_cores=NC, num_subcores=NS)
  lanes_per_row = width // 16

  def body(feat, srcs, dsts, out, sidx, didx0, didx1,
           rows0, rows1, acc, gsem0, gsem1, ssem0, ssem1, dsem0, dsem1):
    c = lax.axis_index("c")
    s = lax.axis_index("s")
    wid = c * NS + s
    ebase = wid * EPW

    pltpu.sync_copy(srcs.at[pl.ds(ebase, EPW)], sidx)

    def dload(i, didx, dsem):
      return pltpu.async_copy(dsts.at[pl.ds(ebase + i * CH, CH)], didx, dsem)

    def dwait(didx, dsem):
      pltpu.make_async_copy(dsts.at[pl.ds(0, CH)], didx, dsem).wait()

    zv = jnp.zeros((16,), jnp.float32)

    def zr(i, carry):
      rows0[i // lanes_per_row, pl.ds((i % lanes_per_row) * 16, 16)] = zv
      return carry

    lax.fori_loop(0, CH * lanes_per_row, zr, 0)
    rbase = s * RPT

    def zc(i, carry):
      pltpu.async_copy(rows0, acc.at[pl.ds(rbase + i * CH, CH)], ssem0)
      return carry

    lax.fori_loop(0, RPT // CH, zc, 0)

    def zw(i, carry):
      pltpu.make_async_copy(rows0, acc.at[pl.ds(rbase, CH)], ssem0).wait()
      return carry

    lax.fori_loop(0, RPT // CH, zw, 0)
    plsc.subcore_barrier()

    def gather(i, rows, gsem):
      return pltpu.async_copy(feat.at[sidx.at[pl.ds(i * CH, CH)]], rows, gsem)

    def gwait(rows, gsem):
      pltpu.make_async_copy(feat.at[sidx.at[pl.ds(0, CH)]], rows, gsem).wait()

    def swait(rows, didx, ssem):
      pltpu.make_async_copy(rows, acc.at[didx], ssem).wait()

    gather(0, rows0, gsem0)
    dload(0, didx0, dsem0)

    def pair(p, carry):
      i0 = 2 * p
      i1 = i0 + 1

      @pl.when(p > 0)
      def _():
        swait(rows1, didx1, ssem1)

      gather(i1, rows1, gsem1)
      dload(i1, didx1, dsem1)
      gwait(rows0, gsem0)
      dwait(didx0, dsem0)
      pltpu.async_copy(rows0, acc.at[didx0], ssem0, add=True)
      swait(rows0, didx0, ssem0)
      gather(i1 + 1, rows0, gsem0)
      dload(i1 + 1, didx0, dsem0)
      gwait(rows1, gsem1)
      dwait(didx1, dsem1)
      pltpu.async_copy(rows1, acc.at[didx1], ssem1, add=True)
      return carry

    lax.fori_loop(0, NCHUNK // 2, pair, 0)
    swait(rows1, didx1, ssem1)
    gwait(rows0, gsem0)
    dwait(didx0, dsem0)
    pltpu.sync_copy(rows0, acc.at[didx0], add=True)
    plsc.subcore_barrier()

    pltpu.sync_copy(acc.at[pl.ds(rbase, RPT)], out.at[c, pl.ds(rbase, RPT)])

  return pl.kernel(
      body,
      out_type=jax.ShapeDtypeStruct((NC, NPAD, width), jnp.float32),
      mesh=mesh,
      scratch_types=[
          pltpu.VMEM((EPW,), jnp.int32),
          pltpu.VMEM((CH,), jnp.int32),
          pltpu.VMEM((CH,), jnp.int32),
          pltpu.VMEM((CH, width), jnp.float32),
          pltpu.VMEM((CH, width), jnp.float32),
          pltpu.VMEM_SHARED((NPAD, width), jnp.float32),
          pltpu.SemaphoreType.DMA,
          pltpu.SemaphoreType.DMA,
          pltpu.SemaphoreType.DMA,
          pltpu.SemaphoreType.DMA,
          pltpu.SemaphoreType.DMA,
          pltpu.SemaphoreType.DMA,
      ],
      compiler_params=pltpu.CompilerParams(use_tc_tiling_on_sc=(width % 128 == 0)),
      name=f"sage_sc_agg_{width}",
  )


_DN = (((1,), (1,)), ((), ()))


def _pre_body(x_ref, wl_ref, wr_ref, b_ref, xlp_ref, xr_ref):
  xlp_ref[:, :DF] = lax.dot_general(
      x_ref[...], wl_ref[...], _DN, preferred_element_type=jnp.float32)
  col = lax.broadcasted_iota(jnp.int32, (NN, DP - DF), 1)
  xlp_ref[:, DF:] = jnp.where(col == 0, 1.0, 0.0)
  xr_ref[...] = lax.dot_general(
      x_ref[...], wr_ref[...], _DN, preferred_element_type=jnp.float32
  ) + b_ref[...]


def _mid_body(acc_ref, xr_ref, wl_ref, wr_ref, b_ref,
              h1l_ref, h1r_ref, dinv_ref):
  accs = acc_ref[0, :NN, :] + acc_ref[1, :NN, :]
  dinv = 1.0 / jnp.maximum(accs[:, DF:DF + 1], 1.0)
  h1 = jnp.maximum(accs[:, :DF] * dinv + xr_ref[...], 0.0)
  h1l_ref[...] = lax.dot_general(
      h1, wl_ref[...], _DN, preferred_element_type=jnp.float32)
  h1r_ref[...] = lax.dot_general(
      h1, wr_ref[...], _DN, preferred_element_type=jnp.float32) + b_ref[...]
  dinv_ref[...] = dinv


def _post_body(acc_ref, dinv_ref, h1r_ref, out_ref):
  accs = acc_ref[0, :NN, :] + acc_ref[1, :NN, :]
  out_ref[...] = accs * dinv_ref[...] + h1r_ref[...]


def kernel(x, edge_index, W1l, b1, W1r, W2l, b2, W2r):
  src = edge_index[0]
  dst = edge_index[1]
  f32 = jnp.float32
  sds = jax.ShapeDtypeStruct

  xlp, xr = pl.pallas_call(
      _pre_body, out_shape=[sds((NN, DP), f32), sds((NN, DF), f32)],
  )(x, W1l, W1r, b1.reshape(1, DF))

  acc1 = _make_agg(DP)(xlp, src, dst)

  h1l, h1r, dinv = pl.pallas_call(
      _mid_body,
      out_shape=[sds((NN, DF), f32), sds((NN, DF), f32), sds((NN, 1), f32)],
  )(acc1, xr, W2l, W2r, b2.reshape(1, DF))

  acc2 = _make_agg(DF)(h1l, src, dst)

  h2 = pl.pallas_call(
      _post_body, out_shape=sds((NN, DF), f32))(acc2, dinv, h1r)

  return h2


# speedup vs baseline: 1.0957x; 1.0907x over previous
import functools

import jax
import jax.numpy as jnp
from jax import lax
from jax.experimental import pallas as pl
from jax.experimental.pallas import tpu as pltpu
from jax.experimental.pallas import tpu_sc as plsc

NN = 10000
NE = 320000
DF = 128
NC = 2
NS = 16
NW = NC * NS
EPW = NE // NW
CH = 80
NCHUNK = EPW // CH
NPAD = 10240
RPT = NPAD // NS


def _make_agg(want_deg):
  mesh = plsc.VectorSubcoreMesh(
      core_axis_name="c", subcore_axis_name="s", num_cores=NC, num_subcores=NS)
  lanes_per_row = DF // 16

  def body(feat, srcs, dsts, *refs):
    if want_deg:
      (out, dout, sidx, didx0, didx1, rows0, rows1, ones, acc, deg1,
       gsem0, gsem1, ssem0, ssem1, dsem0, dsem1, qsem0, qsem1) = refs
    else:
      (out, sidx, didx0, didx1, rows0, rows1, acc,
       gsem0, gsem1, ssem0, ssem1, dsem0, dsem1) = refs
      dout = ones = deg1 = qsem0 = qsem1 = None
    c = lax.axis_index("c")
    s = lax.axis_index("s")
    wid = c * NS + s
    ebase = wid * EPW

    pltpu.sync_copy(srcs.at[pl.ds(ebase, EPW)], sidx)

    def dload(i, didx, dsem):
      return pltpu.async_copy(dsts.at[pl.ds(ebase + i * CH, CH)], didx, dsem)

    def dwait(didx, dsem):
      pltpu.make_async_copy(dsts.at[pl.ds(0, CH)], didx, dsem).wait()

    zv = jnp.zeros((16,), jnp.float32)

    def zr(i, carry):
      rows0[i // lanes_per_row, pl.ds((i % lanes_per_row) * 16, 16)] = zv
      return carry

    lax.fori_loop(0, CH * lanes_per_row, zr, 0)
    rbase = s * RPT

    def zc(i, carry):
      pltpu.async_copy(rows0, acc.at[pl.ds(rbase + i * CH, CH)], ssem0)
      return carry

    lax.fori_loop(0, RPT // CH, zc, 0)

    def zw(i, carry):
      pltpu.make_async_copy(rows0, acc.at[pl.ds(rbase, CH)], ssem0).wait()
      return carry

    lax.fori_loop(0, RPT // CH, zw, 0)

    if want_deg:
      ov = jnp.ones((16,), jnp.float32)

      def fo(i, carry):
        ones[pl.ds(i * 16, 16)] = ov
        return carry

      lax.fori_loop(0, CH // 16, fo, 0)

      def zd(i, carry):
        pltpu.async_copy(
            rows0.at[0], deg1.at[pl.ds(rbase + i * DF, DF)], ssem1)
        return carry

      lax.fori_loop(0, RPT // DF, zd, 0)

      def zdw(i, carry):
        pltpu.make_async_copy(rows0.at[0], deg1.at[pl.ds(0, DF)], ssem1).wait()
        return carry

      lax.fori_loop(0, RPT // DF, zdw, 0)

    plsc.subcore_barrier()

    def gather(i, rows, gsem):
      return pltpu.async_copy(feat.at[sidx.at[pl.ds(i * CH, CH)]], rows, gsem)

    def gwait(rows, gsem):
      pltpu.make_async_copy(feat.at[sidx.at[pl.ds(0, CH)]], rows, gsem).wait()

    def swait(rows, didx, ssem):
      pltpu.make_async_copy(rows, acc.at[didx], ssem).wait()

    def qfire(didx, qsem):
      if want_deg:
        pltpu.async_copy(ones, deg1.at[didx], qsem, add=True)

    def qwait(didx, qsem):
      if want_deg:
        pltpu.make_async_copy(ones, deg1.at[didx], qsem).wait()

    gather(0, rows0, gsem0)
    dload(0, didx0, dsem0)

    def pair(p, carry):
      i0 = 2 * p
      i1 = i0 + 1

      @pl.when(p > 0)
      def _():
        swait(rows1, didx1, ssem1)
        qwait(didx1, qsem1)

      gather(i1, rows1, gsem1)
      dload(i1, didx1, dsem1)
      gwait(rows0, gsem0)
      dwait(didx0, dsem0)
      pltpu.async_copy(rows0, acc.at[didx0], ssem0, add=True)
      qfire(didx0, qsem0)
      swait(rows0, didx0, ssem0)
      qwait(didx0, qsem0)
      gather(i1 + 1, rows0, gsem0)
      dload(i1 + 1, didx0, dsem0)
      gwait(rows1, gsem1)
      dwait(didx1, dsem1)
      pltpu.async_copy(rows1, acc.at[didx1], ssem1, add=True)
      qfire(didx1, qsem1)
      return carry

    lax.fori_loop(0, NCHUNK // 2, pair, 0)
    swait(rows1, didx1, ssem1)
    qwait(didx1, qsem1)
    gwait(rows0, gsem0)
    dwait(didx0, dsem0)
    pltpu.sync_copy(rows0, acc.at[didx0], add=True)
    if want_deg:
      pltpu.sync_copy(ones, deg1.at[didx0], add=True)
    plsc.subcore_barrier()

    pltpu.sync_copy(acc.at[pl.ds(rbase, RPT)], out.at[c, pl.ds(rbase, RPT)])
    if want_deg:
      pltpu.sync_copy(
          deg1.at[pl.ds(rbase, RPT)], dout.at[pl.ds(c * NPAD + rbase, RPT)])

  if want_deg:
    out_type = [
        jax.ShapeDtypeStruct((NC, NPAD, DF), jnp.float32),
        jax.ShapeDtypeStruct((NC * NPAD,), jnp.float32),
    ]
  else:
    out_type = jax.ShapeDtypeStruct((NC, NPAD, DF), jnp.float32)
  scratch = [
      pltpu.VMEM((EPW,), jnp.int32),
      pltpu.VMEM((CH,), jnp.int32),
      pltpu.VMEM((CH,), jnp.int32),
      pltpu.VMEM((CH, DF), jnp.float32),
      pltpu.VMEM((CH, DF), jnp.float32),
  ]
  if want_deg:
    scratch.append(pltpu.VMEM((CH,), jnp.float32))
  scratch.append(pltpu.VMEM_SHARED((NPAD, DF), jnp.float32))
  if want_deg:
    scratch.append(pltpu.VMEM_SHARED((NPAD,), jnp.float32))
  scratch += [pltpu.SemaphoreType.DMA] * (8 if want_deg else 6)
  return pl.kernel(
      body,
      out_type=out_type,
      mesh=mesh,
      scratch_types=scratch,
      compiler_params=pltpu.CompilerParams(use_tc_tiling_on_sc=True),
      name=f"sage_sc_agg{'_deg' if want_deg else ''}",
  )


_DN = (((1,), (1,)), ((), ()))


def _pre_body(x_ref, wl_ref, wr_ref, b_ref, xl_ref, xr_ref):
  xl_ref[...] = lax.dot_general(
      x_ref[...], wl_ref[...], _DN, preferred_element_type=jnp.float32)
  xr_ref[...] = lax.dot_general(
      x_ref[...], wr_ref[...], _DN, preferred_element_type=jnp.float32
  ) + b_ref[...]


def _mid_body(acc_ref, deg_ref, xr_ref, wl_ref, wr_ref, b_ref,
              h1l_ref, h1r_ref, dinv_ref):
  accs = acc_ref[0, :NN, :] + acc_ref[1, :NN, :]
  degs = deg_ref[0, :NN, :] + deg_ref[1, :NN, :]
  dinv = 1.0 / jnp.maximum(degs, 1.0)
  h1 = jnp.maximum(accs * dinv + xr_ref[...], 0.0)
  h1l_ref[...] = lax.dot_general(
      h1, wl_ref[...], _DN, preferred_element_type=jnp.float32)
  h1r_ref[...] = lax.dot_general(
      h1, wr_ref[...], _DN, preferred_element_type=jnp.float32) + b_ref[...]
  dinv_ref[...] = dinv


def _post_body(acc_ref, dinv_ref, h1r_ref, out_ref):
  accs = acc_ref[0, :NN, :] + acc_ref[1, :NN, :]
  out_ref[...] = accs * dinv_ref[...] + h1r_ref[...]


def kernel(x, edge_index, W1l, b1, W1r, W2l, b2, W2r):
  src = edge_index[0]
  dst = edge_index[1]
  f32 = jnp.float32
  sds = jax.ShapeDtypeStruct

  xl, xr = pl.pallas_call(
      _pre_body, out_shape=[sds((NN, DF), f32), sds((NN, DF), f32)],
  )(x, W1l, W1r, b1.reshape(1, DF))

  acc1, deg = _make_agg(True)(xl, src, dst)
  degc = deg.reshape(NC, NPAD, 1)

  h1l, h1r, dinv = pl.pallas_call(
      _mid_body,
      out_shape=[sds((NN, DF), f32), sds((NN, DF), f32), sds((NN, 1), f32)],
  )(acc1, degc, xr, W2l, W2r, b2.reshape(1, DF))

  acc2 = _make_agg(False)(h1l, src, dst)

  h2 = pl.pallas_call(
      _post_body, out_shape=sds((NN, DF), f32))(acc2, dinv, h1r)

  return h2
